# Initial kernel scaffold; baseline (speedup 1.0000x reference)
import jax
import jax.numpy as jnp
from jax.experimental import pallas as pl


def kernel(experience_embeddings, associative_weights, experience, temporal_context, position):
    raise NotImplementedError("write your pallas kernel here")


# speedup vs baseline: 2.2417x; 2.2417x over previous
import jax
import jax.numpy as jnp
from jax import lax
from jax.experimental import pallas as pl
from jax.experimental.pallas import tpu as pltpu

N = 8192
D = 128
BLK = 256


def _emb_sims_kernel(pos_ref, e_ref, emb_ref, new_emb_ref, sims_ref):
    pos = pos_ref[0]
    E = emb_ref[...]
    ev = e_ref[...]
    dots = lax.dot_general(E, ev, (((1,), (1,)), ((), ())),
                           preferred_element_type=jnp.float32)
    n2 = jnp.sum(E * E, axis=1, keepdims=True)
    sims_ref[...] = dots / (jnp.sqrt(n2) + 1e-8)
    rows = lax.broadcasted_iota(jnp.int32, (N, D), 0)
    new_emb_ref[...] = jnp.where(rows == pos, ev, E)


def _weights_kernel(pos_ref, w_ref, sims_col_ref, sims_row_ref, out_ref):
    i = pl.program_id(0)
    pos = pos_ref[0]
    x = w_ref[...]
    cols = lax.broadcasted_iota(jnp.int32, (BLK, N), 1)
    x = jnp.where(cols == pos, sims_col_ref[...], x)
    rows = lax.broadcasted_iota(jnp.int32, (BLK, N), 0) + i * BLK
    out_ref[...] = jnp.where(rows == pos, sims_row_ref[...], x)


def kernel(experience_embeddings, associative_weights, experience,
           temporal_context, position):
    del temporal_context
    pos = jnp.asarray(position, jnp.int32).reshape(1)
    e2 = experience.reshape(1, D)

    new_emb, sims = pl.pallas_call(
        _emb_sims_kernel,
        out_shape=(jax.ShapeDtypeStruct((N, D), jnp.float32),
                   jax.ShapeDtypeStruct((N, 1), jnp.float32)),
        in_specs=[pl.BlockSpec(memory_space=pltpu.SMEM),
                  pl.BlockSpec((1, D), lambda: (0, 0)),
                  pl.BlockSpec((N, D), lambda: (0, 0))],
        out_specs=(pl.BlockSpec((N, D), lambda: (0, 0)),
                   pl.BlockSpec((N, 1), lambda: (0, 0))),
    )(pos, e2, experience_embeddings)

    sims_row = sims.reshape(1, N)

    new_w = pl.pallas_call(
        _weights_kernel,
        grid=(N // BLK,),
        out_shape=jax.ShapeDtypeStruct((N, N), jnp.float32),
        in_specs=[pl.BlockSpec(memory_space=pltpu.SMEM),
                  pl.BlockSpec((BLK, N), lambda i: (i, 0)),
                  pl.BlockSpec((BLK, 1), lambda i: (i, 0)),
                  pl.BlockSpec((1, N), lambda i: (0, 0))],
        out_specs=pl.BlockSpec((BLK, N), lambda i: (i, 0)),
    )(pos, associative_weights, sims, sims_row)

    return (new_emb, new_w)


# speedup vs baseline: 2.3455x; 1.0463x over previous
import jax
import jax.numpy as jnp
from jax import lax
from jax.experimental import pallas as pl
from jax.experimental.pallas import tpu as pltpu

N = 8192
D = 128
BLK = 256


def _fused_kernel(pos_ref, e_ref, emb_ref, w_ref, new_emb_ref, out_ref,
                  sc_ref, sr_ref):
    i = pl.program_id(0)
    pos = pos_ref[0]

    @pl.when(i == 0)
    def _():
        E = emb_ref[...]
        ev = e_ref[...]
        dots_c = lax.dot_general(E, ev, (((1,), (1,)), ((), ())),
                                 preferred_element_type=jnp.float32)
        n2_c = jnp.sum(E * E, axis=1, keepdims=True)
        sc_ref[...] = dots_c / (jnp.sqrt(n2_c) + 1e-8)
        dots_r = lax.dot_general(ev, E, (((1,), (1,)), ((), ())),
                                 preferred_element_type=jnp.float32)
        ones = jnp.ones((1, D), jnp.float32)
        n2_r = lax.dot_general(ones, E * E, (((1,), (1,)), ((), ())),
                               preferred_element_type=jnp.float32)
        sr_ref[...] = dots_r / (jnp.sqrt(n2_r) + 1e-8)
        rows0 = lax.broadcasted_iota(jnp.int32, (N, D), 0)
        new_emb_ref[...] = jnp.where(rows0 == pos, ev, E)

    x = w_ref[...]
    cols = lax.broadcasted_iota(jnp.int32, (BLK, N), 1)
    x = jnp.where(cols == pos, sc_ref[pl.ds(i * BLK, BLK), :], x)
    rows = lax.broadcasted_iota(jnp.int32, (BLK, N), 0) + i * BLK
    out_ref[...] = jnp.where(rows == pos, sr_ref[...], x)


def kernel(experience_embeddings, associative_weights, experience,
           temporal_context, position):
    del temporal_context
    pos = jnp.asarray(position, jnp.int32).reshape(1)
    e2 = experience.reshape(1, D)

    new_emb, new_w = pl.pallas_call(
        _fused_kernel,
        grid=(N // BLK,),
        out_shape=(jax.ShapeDtypeStruct((N, D), jnp.float32),
                   jax.ShapeDtypeStruct((N, N), jnp.float32)),
        in_specs=[pl.BlockSpec(memory_space=pltpu.SMEM),
                  pl.BlockSpec((1, D), lambda i: (0, 0)),
                  pl.BlockSpec((N, D), lambda i: (0, 0)),
                  pl.BlockSpec((BLK, N), lambda i: (i, 0))],
        out_specs=(pl.BlockSpec((N, D), lambda i: (0, 0)),
                   pl.BlockSpec((BLK, N), lambda i: (i, 0))),
        scratch_shapes=[pltpu.VMEM((N, 1), jnp.float32),
                        pltpu.VMEM((1, N), jnp.float32)],
    )(pos, e2, experience_embeddings, associative_weights)

    return (new_emb, new_w)
